# Initial kernel scaffold; baseline (speedup 1.0000x reference)
import jax
import jax.numpy as jnp
from jax.experimental import pallas as pl


def kernel(state_inp, tar_scores, W_sp1, b_sp1, W_sp2, b_sp2, emb_table, W_emb, b_emb, W_m1, b_m1, W_m2, b_m2, W_a1, b_a1, W_a2, b_a2):
    raise NotImplementedError("write your pallas kernel here")


# speedup vs baseline: 6.4310x; 6.4310x over previous
import jax
import jax.numpy as jnp
from jax.experimental import pallas as pl
from jax.experimental.pallas import tpu as pltpu

BS = 4000
NOBJ = 25
NP = 32
K = 16
HID = 32
EMB = 16
MAX_ACTION = 1.0
LOG_STD_MIN, LOG_STD_MAX = -5.0, 2.0
B = 40
BN = B * NP


def _mm(x, w):
    return jax.lax.dot_general(
        x, w, (((1,), (1,)), ((), ())), preferred_element_type=jnp.float32)


def _block_kernel(nin_ref, cat_ref, px_ref, py_ref,
                  Wsp1_ref, bsp1_ref, Wsp2_ref, bsp2_ref,
                  emb_ref, Wemb_ref, bemb_ref,
                  Wm1_ref, bm1_ref, Wm2_ref, bm2_ref,
                  Wa1_ref, ba1_ref, Wa2_ref, ba2_ref,
                  out_ref):
    f32 = jnp.float32

    nin = nin_ref[...]
    col4 = jax.lax.broadcasted_iota(jnp.int32, (1, 4), 1)
    spin = jnp.where(col4 < 2, nin, jnp.tanh(nin))
    se = _mm(jnp.tanh(_mm(spin, Wsp1_ref[...]) + bsp1_ref[...]),
             Wsp2_ref[...]) + bsp2_ref[...]

    te = _mm(jnp.tanh(emb_ref[...]), Wemb_ref[...])
    i3 = jax.lax.broadcasted_iota(jnp.int32, (1, 3), 1).astype(f32)
    oh = (cat_ref[...] == i3).astype(f32)
    ce = _mm(oh, te.T) + bemb_ref[...]

    fs = jnp.tanh(se)
    fc = jnp.tanh(ce)

    Wm1 = Wm1_ref[...]
    Wa = Wm1[:, :HID + EMB]
    Wb = Wm1[:, HID + EMB:]
    Uw = Wa - Wb
    u = _mm(fs, Uw[:, :HID]) + _mm(fc, Uw[:, HID:]) + bm1_ref[...]
    v = _mm(fs, Wb[:, :HID]) + _mm(fc, Wb[:, HID:])

    px = px_ref[...]
    py = py_ref[...]
    dx = px[:, :, None] - px[:, None, :]
    dy = py[:, :, None] - py[:, None, :]
    d2 = dx * dx + dy * dy
    ii = jax.lax.broadcasted_iota(jnp.int32, (NP, NP), 0)
    jj = jax.lax.broadcasted_iota(jnp.int32, (NP, NP), 1)
    d2 = d2 + jnp.where(ii == jj, f32(1e10), f32(0.0))[None]
    a_j = d2[:, :, :, None]
    a_k = d2[:, :, None, :]
    jj4 = jax.lax.broadcasted_iota(jnp.int32, (1, 1, NP, NP), 2)
    kk4 = jax.lax.broadcasted_iota(jnp.int32, (1, 1, NP, NP), 3)
    sel = (a_k < a_j) | ((a_k == a_j) & (kk4 < jj4))
    rank = jnp.sum(sel.astype(f32), axis=3)
    nmask = rank < f32(K)

    u3 = u.reshape(B, NP, HID)
    v3 = v.reshape(B, NP, HID)
    h = jnp.tanh(u3[:, :, None, :] + v3[:, None, :, :])
    msg = _mm(h.reshape(B * NP * NP, HID), Wm2_ref[...]) + bm2_ref[...]
    msg = msg.reshape(B, NP, NP, HID)
    msg = jnp.where(nmask[:, :, :, None], msg, f32(-1e30))
    agg = jnp.max(msg, axis=2).reshape(BN, HID)

    x = jnp.tanh(agg)
    h1 = jnp.tanh(_mm(x, Wa1_ref[...]) + ba1_ref[...])
    out = _mm(h1, Wa2_ref[...]) + ba2_ref[...]
    mu = MAX_ACTION * jnp.tanh(out[:, 0:2])
    ls = jnp.tanh(out[:, 2:4])
    ls = LOG_STD_MIN + 0.5 * (LOG_STD_MAX - LOG_STD_MIN) * (ls + 1.0)
    std = jnp.exp(ls)
    out_ref[...] = jnp.concatenate([mu, std], axis=1)


def kernel(state_inp, tar_scores, W_sp1, b_sp1, W_sp2, b_sp2,
           emb_table, W_emb, b_emb, W_m1, b_m1, W_m2, b_m2,
           W_a1, b_a1, W_a2, b_a2):
    f32 = jnp.float32
    s3 = state_inp.reshape(BS, NOBJ, 3)
    pad_n = ((0, 0), (0, NP - NOBJ))
    px = jnp.pad(s3[:, :, 0], pad_n, constant_values=1e4)
    py = jnp.pad(s3[:, :, 1], pad_n, constant_values=1e4)
    nin = jnp.concatenate([s3[:, :, :2], tar_scores.reshape(BS, NOBJ, 2)],
                          axis=-1)
    nin = jnp.pad(nin, ((0, 0), (0, NP - NOBJ), (0, 0))).reshape(BS * NP, 4)
    cat = jnp.pad(s3[:, :, 2], pad_n).reshape(BS * NP, 1)

    row = lambda b: b.reshape(1, -1).astype(f32)
    grid = (BS // B,)
    node_spec = lambda w: pl.BlockSpec((BN, w), lambda i: (i, 0))
    samp_spec = pl.BlockSpec((B, NP), lambda i: (i, 0))
    full = lambda a: pl.BlockSpec(a.shape, lambda i: (0,) * a.ndim)

    weights = [W_sp1, row(b_sp1), W_sp2, row(b_sp2),
               emb_table, W_emb, row(b_emb),
               W_m1, row(b_m1), W_m2, row(b_m2),
               W_a1, row(b_a1), W_a2, row(b_a2)]

    res = pl.pallas_call(
        _block_kernel,
        grid=grid,
        in_specs=[node_spec(4), node_spec(1), samp_spec, samp_spec]
                 + [full(w) for w in weights],
        out_specs=pl.BlockSpec((BN, 4), lambda i: (i, 0)),
        out_shape=jax.ShapeDtypeStruct((BS * NP, 4), f32),
        compiler_params=pltpu.CompilerParams(
            dimension_semantics=("arbitrary",)),
    )(nin, cat, px, py, *weights)

    res = res.reshape(BS, NP, 4)[:, :NOBJ, :]
    mu = res[:, :, 0:2].reshape(BS, 2 * NOBJ)
    std = res[:, :, 2:4].reshape(BS, 2 * NOBJ)
    return (mu, std)


# speedup vs baseline: 10.3741x; 1.6131x over previous
import jax
import jax.numpy as jnp
from jax.experimental import pallas as pl
from jax.experimental.pallas import tpu as pltpu

BS = 4000
NOBJ = 25
NP = 32
K = 16
HID = 32
EMB = 16
MAX_ACTION = 1.0
LOG_STD_MIN, LOG_STD_MAX = -5.0, 2.0
B = 40
BN = B * NP


def _mm(x, w):
    return jax.lax.dot_general(
        x, w, (((1,), (1,)), ((), ())), preferred_element_type=jnp.float32)


def _block_kernel(nin_ref, cat_ref, px_ref, py_ref,
                  Wsp1_ref, bsp1_ref, Wsp2_ref, bsp2_ref,
                  emb_ref, Wemb_ref, bemb_ref,
                  Wm1_ref, bm1_ref, Wm2_ref, bm2_ref,
                  Wa1_ref, ba1_ref, Wa2_ref, ba2_ref,
                  out_ref):
    f32 = jnp.float32

    nin = nin_ref[...]
    col4 = jax.lax.broadcasted_iota(jnp.int32, (1, 4), 1)
    spin = jnp.where(col4 < 2, nin, jnp.tanh(nin))
    se = _mm(jnp.tanh(_mm(spin, Wsp1_ref[...]) + bsp1_ref[...]),
             Wsp2_ref[...]) + bsp2_ref[...]

    te = _mm(jnp.tanh(emb_ref[...]), Wemb_ref[...])
    i3 = jax.lax.broadcasted_iota(jnp.int32, (1, 3), 1).astype(f32)
    oh = (cat_ref[...] == i3).astype(f32)
    ce = _mm(oh, te.T) + bemb_ref[...]

    fs = jnp.tanh(se)
    fc = jnp.tanh(ce)

    Wm1 = Wm1_ref[...]
    Wa = Wm1[:, :HID + EMB]
    Wb = Wm1[:, HID + EMB:]
    Uw = Wa - Wb
    u = _mm(fs, Uw[:, :HID]) + _mm(fc, Uw[:, HID:]) + bm1_ref[...]
    v = _mm(fs, Wb[:, :HID]) + _mm(fc, Wb[:, HID:])

    px = px_ref[...]
    py = py_ref[...]
    dx = px[:, :, None] - px[:, None, :]
    dy = py[:, :, None] - py[:, None, :]
    d2 = dx * dx + dy * dy
    ii = jax.lax.broadcasted_iota(jnp.int32, (NP, NP), 0)
    jj = jax.lax.broadcasted_iota(jnp.int32, (NP, NP), 1)
    d2 = d2 + jnp.where(ii == jj, f32(1e10), f32(0.0))[None]
    bits = jax.lax.bitcast_convert_type(d2, jnp.int32)
    key = (bits & jnp.int32(-32)) | jj[None]

    G = NP // 4
    L = 4 * HID
    u3 = u.reshape(B, NP, HID)
    v3 = v.reshape(B, NP, HID)

    def slabs(x):
        return jnp.concatenate([x[:, s * G:(s + 1) * G, :] for s in range(4)],
                               axis=2)

    keyI = slabs(key)
    keyJ = jnp.concatenate(
        [jnp.broadcast_to(key[:, s * G:(s + 1) * G, :, None], (B, G, NP, HID))
         for s in range(4)], axis=3)
    sel = (keyI[:, :, None, :] < keyJ).astype(f32)
    r128 = jax.lax.broadcasted_iota(jnp.int32, (1, L), 1)
    c128 = jax.lax.broadcasted_iota(jnp.int32, (L, 1), 0)
    blkmask = (c128 // HID == r128 // HID).astype(f32)
    rank = _mm(sel.reshape(B * G * NP, L), blkmask).reshape(B, G, NP, L)
    pen = jnp.where(rank < f32(K), f32(0.0), f32(-1e30))

    upk = slabs(u3)
    vtl = jnp.concatenate([v3] * 4, axis=2)
    h = jnp.tanh(upk[:, :, None, :] + vtl[:, None, :, :])
    Wblk = jnp.concatenate([jnp.concatenate([Wm2_ref[...]] * 4, axis=0)] * 4,
                           axis=1) * blkmask
    b4 = jnp.concatenate([bm2_ref[...]] * 4, axis=1)
    msg = _mm(h.reshape(B * G * NP, L), Wblk) + b4
    msg = msg.reshape(B, G, NP, L) + pen
    agg = jnp.max(msg, axis=2)
    x3 = jnp.concatenate([agg[:, :, s * HID:(s + 1) * HID] for s in range(4)],
                         axis=1)
    agg = x3.reshape(BN, HID)

    x = jnp.tanh(agg)
    h1 = jnp.tanh(_mm(x, Wa1_ref[...]) + ba1_ref[...])
    out = _mm(h1, Wa2_ref[...]) + ba2_ref[...]
    mu = MAX_ACTION * jnp.tanh(out[:, 0:2])
    ls = jnp.tanh(out[:, 2:4])
    ls = LOG_STD_MIN + 0.5 * (LOG_STD_MAX - LOG_STD_MIN) * (ls + 1.0)
    std = jnp.exp(ls)
    out_ref[...] = jnp.concatenate([mu, std], axis=1)


def kernel(state_inp, tar_scores, W_sp1, b_sp1, W_sp2, b_sp2,
           emb_table, W_emb, b_emb, W_m1, b_m1, W_m2, b_m2,
           W_a1, b_a1, W_a2, b_a2):
    f32 = jnp.float32
    s3 = state_inp.reshape(BS, NOBJ, 3)
    pad_n = ((0, 0), (0, NP - NOBJ))
    px = jnp.pad(s3[:, :, 0], pad_n, constant_values=1e4)
    py = jnp.pad(s3[:, :, 1], pad_n, constant_values=1e4)
    nin = jnp.concatenate([s3[:, :, :2], tar_scores.reshape(BS, NOBJ, 2)],
                          axis=-1)
    nin = jnp.pad(nin, ((0, 0), (0, NP - NOBJ), (0, 0))).reshape(BS * NP, 4)
    cat = jnp.pad(s3[:, :, 2], pad_n).reshape(BS * NP, 1)

    row = lambda b: b.reshape(1, -1).astype(f32)
    grid = (BS // B,)
    node_spec = lambda w: pl.BlockSpec((BN, w), lambda i: (i, 0))
    samp_spec = pl.BlockSpec((B, NP), lambda i: (i, 0))
    full = lambda a: pl.BlockSpec(a.shape, lambda i: (0,) * a.ndim)

    weights = [W_sp1, row(b_sp1), W_sp2, row(b_sp2),
               emb_table, W_emb, row(b_emb),
               W_m1, row(b_m1), W_m2, row(b_m2),
               W_a1, row(b_a1), W_a2, row(b_a2)]

    res = pl.pallas_call(
        _block_kernel,
        grid=grid,
        in_specs=[node_spec(4), node_spec(1), samp_spec, samp_spec]
                 + [full(w) for w in weights],
        out_specs=pl.BlockSpec((BN, 4), lambda i: (i, 0)),
        out_shape=jax.ShapeDtypeStruct((BS * NP, 4), f32),
        compiler_params=pltpu.CompilerParams(
            dimension_semantics=("arbitrary",)),
    )(nin, cat, px, py, *weights)

    res = res.reshape(BS, NP, 4)[:, :NOBJ, :]
    mu = res[:, :, 0:2].reshape(BS, 2 * NOBJ)
    std = res[:, :, 2:4].reshape(BS, 2 * NOBJ)
    return (mu, std)


# speedup vs baseline: 12.1574x; 1.1719x over previous
import jax
import jax.numpy as jnp
import numpy as np
from jax.experimental import pallas as pl
from jax.experimental.pallas import tpu as pltpu

BS = 4000
NOBJ = 25
NP = 32
K = 16
HID = 32
EMB = 16
MAX_ACTION = 1.0
LOG_STD_MIN, LOG_STD_MAX = -5.0, 2.0
B = 80
BN = B * NP


def _mm(x, w):
    return jax.lax.dot_general(
        x, w, (((1,), (1,)), ((), ())), preferred_element_type=jnp.float32)


def _block_kernel(nin_ref, cat_ref, px_ref, py_ref,
                  Wsp1_ref, bsp1_ref, Wsp2_ref, bsp2_ref,
                  emb_ref, Wemb_ref, bemb_ref,
                  Wm1_ref, bm1_ref, blkmask_ref, Wblk_ref, b4_ref,
                  Wa1_ref, ba1_ref, Wa2_ref, ba2_ref,
                  out_ref):
    f32 = jnp.float32

    nin = nin_ref[...]
    col4 = jax.lax.broadcasted_iota(jnp.int32, (1, 4), 1)
    spin = jnp.where(col4 < 2, nin, jnp.tanh(nin))
    se = _mm(jnp.tanh(_mm(spin, Wsp1_ref[...]) + bsp1_ref[...]),
             Wsp2_ref[...]) + bsp2_ref[...]

    te = _mm(jnp.tanh(emb_ref[...]), Wemb_ref[...])
    i3 = jax.lax.broadcasted_iota(jnp.int32, (1, 3), 1).astype(f32)
    oh = (cat_ref[...] == i3).astype(f32)
    ce = _mm(oh, te.T) + bemb_ref[...]

    fs = jnp.tanh(se)
    fc = jnp.tanh(ce)

    Wm1 = Wm1_ref[...]
    Wa = Wm1[:, :HID + EMB]
    Wb = Wm1[:, HID + EMB:]
    Uw = Wa - Wb
    u = _mm(fs, Uw[:, :HID]) + _mm(fc, Uw[:, HID:]) + bm1_ref[...]
    v = _mm(fs, Wb[:, :HID]) + _mm(fc, Wb[:, HID:])

    px = px_ref[...]
    py = py_ref[...]
    dx = px[:, :, None] - px[:, None, :]
    dy = py[:, :, None] - py[:, None, :]
    d2 = dx * dx + dy * dy
    ii = jax.lax.broadcasted_iota(jnp.int32, (NP, NP), 0)
    jj = jax.lax.broadcasted_iota(jnp.int32, (NP, NP), 1)
    d2 = d2 + jnp.where(ii == jj, f32(1e10), f32(0.0))[None]
    bits = jax.lax.bitcast_convert_type(d2, jnp.int32)
    key = (bits & jnp.int32(-32)) | jj[None]

    G = NP // 4
    L = 4 * HID
    u3 = u.reshape(B, NP, HID)
    v3 = v.reshape(B, NP, HID)

    def slabs(x):
        return jnp.concatenate([x[:, s * G:(s + 1) * G, :] for s in range(4)],
                               axis=2)

    keyI = slabs(key)
    keyJ = jnp.concatenate(
        [jnp.broadcast_to(key[:, s * G:(s + 1) * G, :, None], (B, G, NP, HID))
         for s in range(4)], axis=3)
    sel = (keyI[:, :, None, :] < keyJ).astype(f32)
    blkmask = blkmask_ref[...]
    rank = _mm(sel.reshape(B * G * NP, L), blkmask).reshape(B, G, NP, L)
    pen = jnp.where(rank < f32(K), f32(0.0), f32(-1e30))

    upk = slabs(u3)
    vtl = jnp.concatenate([v3] * 4, axis=2)
    h = jnp.tanh(upk[:, :, None, :] + vtl[:, None, :, :])
    msg = _mm(h.reshape(B * G * NP, L), Wblk_ref[...]) + b4_ref[...]
    msg = msg.reshape(B, G, NP, L) + pen
    agg = jnp.max(msg, axis=2)
    x3 = jnp.concatenate([agg[:, :, s * HID:(s + 1) * HID] for s in range(4)],
                         axis=1)
    agg = x3.reshape(BN, HID)

    x = jnp.tanh(agg)
    h1 = jnp.tanh(_mm(x, Wa1_ref[...]) + ba1_ref[...])
    out = _mm(h1, Wa2_ref[...]) + ba2_ref[...]
    mu = MAX_ACTION * jnp.tanh(out[:, 0:2])
    ls = jnp.tanh(out[:, 2:4])
    ls = LOG_STD_MIN + 0.5 * (LOG_STD_MAX - LOG_STD_MIN) * (ls + 1.0)
    std = jnp.exp(ls)
    out_ref[...] = jnp.concatenate([mu, std], axis=1)


def kernel(state_inp, tar_scores, W_sp1, b_sp1, W_sp2, b_sp2,
           emb_table, W_emb, b_emb, W_m1, b_m1, W_m2, b_m2,
           W_a1, b_a1, W_a2, b_a2):
    f32 = jnp.float32
    s3 = state_inp.reshape(BS, NOBJ, 3)
    pad_n = ((0, 0), (0, NP - NOBJ))
    px = jnp.pad(s3[:, :, 0], pad_n, constant_values=1e4)
    py = jnp.pad(s3[:, :, 1], pad_n, constant_values=1e4)
    nin = jnp.concatenate([s3[:, :, :2], tar_scores.reshape(BS, NOBJ, 2)],
                          axis=-1)
    nin = jnp.pad(nin, ((0, 0), (0, NP - NOBJ), (0, 0))).reshape(BS * NP, 4)
    cat = jnp.pad(s3[:, :, 2], pad_n).reshape(BS * NP, 1)

    row = lambda b: b.reshape(1, -1).astype(f32)
    L = 4 * HID
    lane = np.arange(L)
    blkmask = (lane[:, None] // HID == lane[None, :] // HID).astype(np.float32)
    Wblk = jnp.tile(W_m2, (4, 4)) * blkmask
    b4 = jnp.tile(b_m2.reshape(1, -1), (1, 4))
    grid = (BS // B,)
    node_spec = lambda w: pl.BlockSpec((BN, w), lambda i: (i, 0))
    samp_spec = pl.BlockSpec((B, NP), lambda i: (i, 0))
    full = lambda a: pl.BlockSpec(a.shape, lambda i: (0,) * a.ndim)

    weights = [W_sp1, row(b_sp1), W_sp2, row(b_sp2),
               emb_table, W_emb, row(b_emb),
               W_m1, row(b_m1), jnp.asarray(blkmask), Wblk, b4,
               W_a1, row(b_a1), W_a2, row(b_a2)]

    res = pl.pallas_call(
        _block_kernel,
        grid=grid,
        in_specs=[node_spec(4), node_spec(1), samp_spec, samp_spec]
                 + [full(w) for w in weights],
        out_specs=pl.BlockSpec((BN, 4), lambda i: (i, 0)),
        out_shape=jax.ShapeDtypeStruct((BS * NP, 4), f32),
        compiler_params=pltpu.CompilerParams(
            dimension_semantics=("arbitrary",)),
    )(nin, cat, px, py, *weights)

    res = res.reshape(BS, NP, 4)[:, :NOBJ, :]
    mu = res[:, :, 0:2].reshape(BS, 2 * NOBJ)
    std = res[:, :, 2:4].reshape(BS, 2 * NOBJ)
    return (mu, std)
